# baseline (speedup 1.0000x reference)
import functools

import jax
import jax.numpy as jnp
from jax import lax
from jax.experimental import pallas as pl
from jax.experimental.pallas import tpu as pltpu
from jax.experimental.pallas import tpu_sc as plsc

N = 10000
NP = 10240
E = 320000
L = 200000
D = 128

NC = 2
NS = 16
NW = NC * NS


_CF = 176
_NF1 = 113
_RM1 = 112


def _segsum_deg_kernel(feats_hbm, src_hbm, dst_hbm, zrows_hbm, ones_hbm,
                       out_hbm, acc_sp,
                       srcv0, rows0, semg0, semi0, srcv1, rows1, semg1, semi1,
                       dstv0, dstv1, dstv2, dstv3, srce, dste):
    cid = lax.axis_index("c")
    sid = lax.axis_index("s")
    n0 = sid * (NP // NS)
    nn = NP // NS
    e0 = sid * (E // NS)

    pltpu.sync_copy(zrows_hbm, acc_sp.at[pl.ds(n0, nn)])
    plsc.subcore_barrier()

    gbufs = ((srcv0, rows0, semg0, semi0), (srcv1, rows1, semg1, semi1))
    dring = (dstv0, dstv1, dstv2, dstv3)

    @pl.when(cid == 0)
    def _seg():
        for c in range(2):
            srcv, rows, semg, _ = gbufs[c % 2]
            base = e0 + c * _CF
            pltpu.sync_copy(src_hbm.at[pl.ds(base, _CF)], srcv)
            pltpu.sync_copy(dst_hbm.at[pl.ds(base, _CF)], dring[c])
            pltpu.async_copy(feats_hbm.at[srcv], rows, semg)

        def body(jo, _):
            for q in range(4):
                c = 4 * jo + q
                b = q % 2
                srcv, rows, semg, semi = gbufs[b]
                pltpu.make_async_copy(feats_hbm.at[srcv], rows, semg).wait()

                @pl.when(c + 2 < _NF1)
                def _idx():
                    base2 = e0 + (c + 2) * _CF
                    pltpu.async_copy(src_hbm.at[pl.ds(base2, _CF)], srcv,
                                     semi)
                    pltpu.async_copy(dst_hbm.at[pl.ds(base2, _CF)],
                                     dring[(q + 2) % 4], semi)

                pltpu.sync_copy(rows, acc_sp.at[dring[q]], add=True)

                @pl.when(c + 2 < _NF1)
                def _fire():
                    base2 = e0 + (c + 2) * _CF
                    pltpu.make_async_copy(src_hbm.at[pl.ds(base2, _CF)],
                                          srcv, semi).wait()
                    pltpu.make_async_copy(dst_hbm.at[pl.ds(base2, _CF)],
                                          dring[(q + 2) % 4], semi).wait()
                    pltpu.async_copy(feats_hbm.at[srcv], rows, semg)

            return 0

        lax.fori_loop(0, (_NF1 - 1) // 4, body, 0)

        pltpu.make_async_copy(feats_hbm.at[srcv0], rows0, semg0).wait()
        pltpu.sync_copy(rows0, acc_sp.at[dring[0]], add=True)

        base = e0 + _NF1 * _CF
        pltpu.sync_copy(src_hbm.at[pl.ds(base, _RM1)], srce)
        pltpu.sync_copy(dst_hbm.at[pl.ds(base, _RM1)], dste)
        pltpu.async_copy(feats_hbm.at[srce], rows0.at[pl.ds(0, _RM1)],
                         semg0).wait()
        pltpu.sync_copy(rows0.at[pl.ds(0, _RM1)], acc_sp.at[dste], add=True)

    @pl.when(cid == 1)
    def _deg():
        pltpu.sync_copy(ones_hbm, rows0)
        pltpu.sync_copy(dst_hbm.at[pl.ds(e0, _CF)], dstv0)

        def body(jo, _):
            for p in range(2):
                c = 2 * jo + p

                @pl.when(c + 1 < _NF1)
                def _idx():
                    base2 = e0 + (c + 1) * _CF
                    pltpu.async_copy(dst_hbm.at[pl.ds(base2, _CF)],
                                     dring[(p + 1) % 2], semi0)

                pltpu.sync_copy(rows0, acc_sp.at[dring[p]], add=True)

                @pl.when(c + 1 < _NF1)
                def _w():
                    base2 = e0 + (c + 1) * _CF
                    pltpu.make_async_copy(dst_hbm.at[pl.ds(base2, _CF)],
                                          dring[(p + 1) % 2], semi0).wait()

            return 0

        lax.fori_loop(0, (_NF1 - 1) // 2, body, 0)

        pltpu.sync_copy(rows0, acc_sp.at[dring[0]], add=True)

        base = e0 + _NF1 * _CF
        pltpu.sync_copy(dst_hbm.at[pl.ds(base, _RM1)], dste)
        pltpu.sync_copy(rows0.at[pl.ds(0, _RM1)], acc_sp.at[dste], add=True)

    plsc.subcore_barrier()
    pltpu.sync_copy(acc_sp.at[pl.ds(n0, nn)], out_hbm.at[cid, pl.ds(n0, nn)])


def _segsum_deg(feats, src, dst):
    zrows = jnp.zeros((NP // NS, D), jnp.float32)
    ones = jnp.ones((_CF, D), jnp.float32)
    mesh = plsc.VectorSubcoreMesh(core_axis_name="c", subcore_axis_name="s")
    fn = functools.partial(
        pl.kernel,
        mesh=mesh,
        out_type=jax.ShapeDtypeStruct((NC, NP, D), jnp.float32),
        scratch_types=[
            pltpu.VMEM_SHARED((NP, D), jnp.float32),
            pltpu.VMEM((_CF,), jnp.int32),
            pltpu.VMEM((_CF, D), jnp.float32),
            pltpu.SemaphoreType.DMA,
            pltpu.SemaphoreType.DMA,
            pltpu.VMEM((_CF,), jnp.int32),
            pltpu.VMEM((_CF, D), jnp.float32),
            pltpu.SemaphoreType.DMA,
            pltpu.SemaphoreType.DMA,
            pltpu.VMEM((_CF,), jnp.int32),
            pltpu.VMEM((_CF,), jnp.int32),
            pltpu.VMEM((_CF,), jnp.int32),
            pltpu.VMEM((_CF,), jnp.int32),
            pltpu.VMEM((_RM1,), jnp.int32),
            pltpu.VMEM((_RM1,), jnp.int32),
        ],
    )(_segsum_deg_kernel)
    return fn(feats, src, dst, zrows, ones)


_CB = 176
_NFULL = 56
_CREM = 144


def _segsum_kernel(feats_hbm, src_hbm, dst_hbm, zrows_hbm, out_hbm, acc_sp,
                   srcv0, rows0, semg0, semi0, srcv1, rows1, semg1, semi1,
                   dstv0, dstv1, dstv2, dstv3, srce, dste):
    cid = lax.axis_index("c")
    sid = lax.axis_index("s")
    wid = cid * NS + sid
    n0 = sid * (NP // NS)
    nn = NP // NS
    e0 = wid * (E // NW)

    pltpu.sync_copy(zrows_hbm, acc_sp.at[pl.ds(n0, nn)])
    plsc.subcore_barrier()

    gbufs = ((srcv0, rows0, semg0, semi0), (srcv1, rows1, semg1, semi1))
    dring = (dstv0, dstv1, dstv2, dstv3)

    for c in range(2):
        srcv, rows, semg, _ = gbufs[c % 2]
        base = e0 + c * _CB
        pltpu.sync_copy(src_hbm.at[pl.ds(base, _CB)], srcv)
        pltpu.sync_copy(dst_hbm.at[pl.ds(base, _CB)], dring[c])
        pltpu.async_copy(feats_hbm.at[srcv], rows, semg)

    def body(jo, _):
        for q in range(4):
            c = 4 * jo + q
            b = q % 2
            srcv, rows, semg, semi = gbufs[b]
            pltpu.make_async_copy(feats_hbm.at[srcv], rows, semg).wait()

            @pl.when(c + 2 < _NFULL)
            def _idx():
                base2 = e0 + (c + 2) * _CB
                pltpu.async_copy(src_hbm.at[pl.ds(base2, _CB)], srcv, semi)
                pltpu.async_copy(dst_hbm.at[pl.ds(base2, _CB)],
                                 dring[(q + 2) % 4], semi)

            pltpu.sync_copy(rows, acc_sp.at[dring[q % 4]], add=True)

            @pl.when(c + 2 < _NFULL)
            def _fire():
                base2 = e0 + (c + 2) * _CB
                pltpu.make_async_copy(src_hbm.at[pl.ds(base2, _CB)], srcv,
                                      semi).wait()
                pltpu.make_async_copy(dst_hbm.at[pl.ds(base2, _CB)],
                                      dring[(q + 2) % 4], semi).wait()
                pltpu.async_copy(feats_hbm.at[srcv], rows, semg)

        return 0

    lax.fori_loop(0, _NFULL // 4, body, 0)

    base = e0 + _NFULL * _CB
    pltpu.sync_copy(src_hbm.at[pl.ds(base, _CREM)], srce)
    pltpu.sync_copy(dst_hbm.at[pl.ds(base, _CREM)], dste)
    pltpu.async_copy(feats_hbm.at[srce], rows0.at[pl.ds(0, _CREM)],
                     semg0).wait()
    pltpu.sync_copy(rows0.at[pl.ds(0, _CREM)], acc_sp.at[dste], add=True)

    plsc.subcore_barrier()
    pltpu.sync_copy(acc_sp.at[pl.ds(n0, nn)], out_hbm.at[cid, pl.ds(n0, nn)])


def _segsum(feats, src, dst):
    zrows = jnp.zeros((NP // NS, D), jnp.float32)
    mesh = plsc.VectorSubcoreMesh(core_axis_name="c", subcore_axis_name="s")
    fn = functools.partial(
        pl.kernel,
        mesh=mesh,
        out_type=jax.ShapeDtypeStruct((NC, NP, D), jnp.float32),
        scratch_types=[
            pltpu.VMEM_SHARED((NP, D), jnp.float32),
            pltpu.VMEM((_CB,), jnp.int32),
            pltpu.VMEM((_CB, D), jnp.float32),
            pltpu.SemaphoreType.DMA,
            pltpu.SemaphoreType.DMA,
            pltpu.VMEM((_CB,), jnp.int32),
            pltpu.VMEM((_CB, D), jnp.float32),
            pltpu.SemaphoreType.DMA,
            pltpu.SemaphoreType.DMA,
            pltpu.VMEM((_CB,), jnp.int32),
            pltpu.VMEM((_CB,), jnp.int32),
            pltpu.VMEM((_CB,), jnp.int32),
            pltpu.VMEM((_CB,), jnp.int32),
            pltpu.VMEM((_CREM,), jnp.int32),
            pltpu.VMEM((_CREM,), jnp.int32),
        ],
    )(_segsum_kernel)
    return fn(feats, src, dst, zrows)


_RB = 1280


def _layer1_body(pref, fref, wlref, wrref, bref, zref, invref):
    p = pref[...]
    inv = 1.0 / jnp.maximum(p[1, :, 0:1], 1.0)
    agg = p[0] * inv
    dn = (((1,), (1,)), ((), ()))
    h = (lax.dot_general(agg, wlref[...], dn,
                         preferred_element_type=jnp.float32)
         + lax.dot_general(fref[...], wrref[...], dn,
                           preferred_element_type=jnp.float32)
         + bref[...])
    zref[...] = jnp.maximum(h, 0.0)
    invref[...] = inv


def _tc_layer1(partials, feats, WlT, WrT, b2d):
    return pl.pallas_call(
        _layer1_body,
        grid=(NP // _RB,),
        in_specs=[
            pl.BlockSpec((NC, _RB, D), lambda i: (0, i, 0)),
            pl.BlockSpec((_RB, D), lambda i: (i, 0)),
            pl.BlockSpec((D, D), lambda i: (0, 0)),
            pl.BlockSpec((D, D), lambda i: (0, 0)),
            pl.BlockSpec((1, D), lambda i: (0, 0)),
        ],
        out_specs=[pl.BlockSpec((_RB, D), lambda i: (i, 0)),
                   pl.BlockSpec((_RB, 1), lambda i: (i, 0))],
        out_shape=[jax.ShapeDtypeStruct((NP, D), jnp.float32),
                   jax.ShapeDtypeStruct((NP, 1), jnp.float32)],
    )(partials, feats, WlT, WrT, b2d)


def _layer2_body(pref, invref, fref, wlref, wrref, bref, zref):
    p = pref[...]
    agg = (p[0] + p[1]) * invref[...]
    dn = (((1,), (1,)), ((), ()))
    h = (lax.dot_general(agg, wlref[...], dn,
                         preferred_element_type=jnp.float32)
         + lax.dot_general(fref[...], wrref[...], dn,
                           preferred_element_type=jnp.float32)
         + bref[...])
    zref[...] = h


def _tc_layer2(partials, inv_col, feats, WlT, WrT, b2d):
    return pl.pallas_call(
        _layer2_body,
        grid=(NP // _RB,),
        in_specs=[
            pl.BlockSpec((NC, _RB, D), lambda i: (0, i, 0)),
            pl.BlockSpec((_RB, 1), lambda i: (i, 0)),
            pl.BlockSpec((_RB, D), lambda i: (i, 0)),
            pl.BlockSpec((D, D), lambda i: (0, 0)),
            pl.BlockSpec((D, D), lambda i: (0, 0)),
            pl.BlockSpec((1, D), lambda i: (0, 0)),
        ],
        out_specs=pl.BlockSpec((_RB, D), lambda i: (i, 0)),
        out_shape=jax.ShapeDtypeStruct((NP, D), jnp.float32),
    )(partials, inv_col, feats, WlT, WrT, b2d)


_CE = 160
_NCH = L // _CE
LP = 200704


def _decode_kernel(z_hbm, ai_hbm, bi_hbm, out_hbm,
                   aidx0, bidx0, arows0, brows0, dots0, sema0, semb0,
                   aidx1, bidx1, arows1, brows1, dots1, sema1, semb1):
    cid = lax.axis_index("c")
    sid = lax.axis_index("s")
    wid = cid * NS + sid

    bufs = ((aidx0, bidx0, arows0, brows0, dots0, sema0, semb0),
            (aidx1, bidx1, arows1, brows1, dots1, sema1, semb1))

    def fire(j, b):
        ch = wid + j * NW

        @pl.when(ch < _NCH)
        def _f():
            aidx, bidx, arows, brows, dots, sema, semb = bufs[b]
            base = ch * _CE
            pltpu.sync_copy(ai_hbm.at[pl.ds(base, _CE)], aidx)
            pltpu.sync_copy(bi_hbm.at[pl.ds(base, _CE)], bidx)
            pltpu.async_copy(z_hbm.at[aidx], arows, sema)
            pltpu.async_copy(z_hbm.at[bidx], brows, semb)

    fire(0, 0)
    fire(1, 1)

    def chunk_body(jo, _):
        for b in range(2):
            j = 2 * jo + b
            ch = wid + j * NW
            ch2 = wid + (j + 2) * NW

            @pl.when(ch < _NCH)
            def _do():
                aidx, bidx, arows, brows, dots, sema, semb = bufs[b]
                base = ch * _CE
                pltpu.make_async_copy(z_hbm.at[aidx], arows, sema).wait()
                pltpu.make_async_copy(z_hbm.at[bidx], brows, semb).wait()

                @pl.when(ch2 < _NCH)
                def _idx():
                    base2 = ch2 * _CE
                    pltpu.async_copy(ai_hbm.at[pl.ds(base2, _CE)], aidx, sema)
                    pltpu.async_copy(bi_hbm.at[pl.ds(base2, _CE)], bidx, semb)

                def pair_body(g, _):
                    i = g * 4
                    accs = [arows[i + u, pl.ds(0, 16)]
                            * brows[i + u, pl.ds(0, 16)] for u in range(4)]
                    for kk in range(1, D // 16):
                        for u in range(4):
                            accs[u] = accs[u] + (
                                arows[i + u, pl.ds(kk * 16, 16)]
                                * brows[i + u, pl.ds(kk * 16, 16)])
                    for u in range(4):
                        dots[i + u] = accs[u]
                    return 0

                lax.fori_loop(0, _CE // 4, pair_body, 0)
                pltpu.sync_copy(dots, out_hbm.at[pl.ds(base, _CE)])

                @pl.when(ch2 < _NCH)
                def _fire2():
                    base2 = ch2 * _CE
                    pltpu.make_async_copy(ai_hbm.at[pl.ds(base2, _CE)], aidx,
                                          sema).wait()
                    pltpu.make_async_copy(bi_hbm.at[pl.ds(base2, _CE)], bidx,
                                          semb).wait()
                    pltpu.async_copy(z_hbm.at[aidx], arows, sema)
                    pltpu.async_copy(z_hbm.at[bidx], brows, semb)

        return 0

    lax.fori_loop(0, (_NCH + NW - 1) // NW // 2, chunk_body, 0)


def _decode_partial(z, ai, bi):
    mesh = plsc.VectorSubcoreMesh(core_axis_name="c", subcore_axis_name="s")
    buf_types = [
        pltpu.VMEM((_CE,), jnp.int32),
        pltpu.VMEM((_CE,), jnp.int32),
        pltpu.VMEM((_CE, D), jnp.float32),
        pltpu.VMEM((_CE, D), jnp.float32),
        pltpu.VMEM((_CE, 16), jnp.float32),
        pltpu.SemaphoreType.DMA,
        pltpu.SemaphoreType.DMA,
    ]
    fn = functools.partial(
        pl.kernel,
        mesh=mesh,
        out_type=jax.ShapeDtypeStruct((LP, 16), jnp.float32),
        scratch_types=buf_types + buf_types,
    )(_decode_kernel)
    return fn(z, ai, bi)


_RF = 4096


def _fold_body(iref, oref):
    oref[...] = jnp.sum(iref[...], axis=1, keepdims=True)


def _fold16(dots16):
    return pl.pallas_call(
        _fold_body,
        grid=(LP // _RF,),
        in_specs=[pl.BlockSpec((_RF, 16), lambda i: (i, 0))],
        out_specs=pl.BlockSpec((_RF, 1), lambda i: (i, 0)),
        out_shape=jax.ShapeDtypeStruct((LP, 1), jnp.float32),
    )(dots16)


@jax.jit
def kernel(x, edge_index, edge_label_index, W1_l, b1, W1_r, W2_l, b2, W2_r):
    src = edge_index[0]
    dst = edge_index[1]
    xp = jnp.pad(x, ((0, NP - N), (0, 0)))

    p1 = _segsum_deg(xp, src, dst)
    z1, inv_col = _tc_layer1(p1, xp, W1_l, W1_r, b1.reshape(1, D))
    p2 = _segsum(z1, src, dst)
    z2 = _tc_layer2(p2, inv_col, z1, W2_l, W2_r, b2.reshape(1, D))

    dots16 = _decode_partial(z2, edge_label_index[0], edge_label_index[1])
    return _fold16(dots16).reshape(LP)[:L]


# speedup vs baseline: 1.0063x; 1.0063x over previous
import functools

import jax
import jax.numpy as jnp
from jax import lax
from jax.experimental import pallas as pl
from jax.experimental.pallas import tpu as pltpu
from jax.experimental.pallas import tpu_sc as plsc

N = 10000
NP = 10240
E = 320000
L = 200000
D = 128

NC = 2
NS = 16
NW = NC * NS


_CF = 176
_NF1 = 113
_RM1 = 112


def _segsum_deg_kernel(feats_hbm, src_hbm, dst_hbm, zrows_hbm, ones_hbm,
                       out_hbm, acc_sp,
                       srcv0, rows0, semg0, semi0, srcv1, rows1, semg1, semi1,
                       dstv0, dstv1, dstv2, dstv3, srce, dste):
    cid = lax.axis_index("c")
    sid = lax.axis_index("s")
    n0 = sid * (NP // NS)
    nn = NP // NS
    e0 = sid * (E // NS)

    pltpu.sync_copy(zrows_hbm, acc_sp.at[pl.ds(n0, nn)])
    plsc.subcore_barrier()

    gbufs = ((srcv0, rows0, semg0, semi0), (srcv1, rows1, semg1, semi1))
    dring = (dstv0, dstv1, dstv2, dstv3)

    @pl.when(cid == 0)
    def _seg():
        for c in range(2):
            srcv, rows, semg, _ = gbufs[c % 2]
            base = e0 + c * _CF
            pltpu.sync_copy(src_hbm.at[pl.ds(base, _CF)], srcv)
            pltpu.sync_copy(dst_hbm.at[pl.ds(base, _CF)], dring[c])
            pltpu.async_copy(feats_hbm.at[srcv], rows, semg)

        def body(jo, _):
            for q in range(4):
                c = 4 * jo + q
                b = q % 2
                srcv, rows, semg, semi = gbufs[b]
                pltpu.make_async_copy(feats_hbm.at[srcv], rows, semg).wait()

                @pl.when(c + 2 < _NF1)
                def _idx():
                    base2 = e0 + (c + 2) * _CF
                    pltpu.async_copy(src_hbm.at[pl.ds(base2, _CF)], srcv,
                                     semi)
                    pltpu.async_copy(dst_hbm.at[pl.ds(base2, _CF)],
                                     dring[(q + 2) % 4], semi)

                pltpu.sync_copy(rows, acc_sp.at[dring[q]], add=True)

                @pl.when(c + 2 < _NF1)
                def _fire():
                    base2 = e0 + (c + 2) * _CF
                    pltpu.make_async_copy(src_hbm.at[pl.ds(base2, _CF)],
                                          srcv, semi).wait()
                    pltpu.make_async_copy(dst_hbm.at[pl.ds(base2, _CF)],
                                          dring[(q + 2) % 4], semi).wait()
                    pltpu.async_copy(feats_hbm.at[srcv], rows, semg)

            return 0

        lax.fori_loop(0, (_NF1 - 1) // 4, body, 0)

        pltpu.make_async_copy(feats_hbm.at[srcv0], rows0, semg0).wait()
        pltpu.sync_copy(rows0, acc_sp.at[dring[0]], add=True)

        base = e0 + _NF1 * _CF
        pltpu.sync_copy(src_hbm.at[pl.ds(base, _RM1)], srce)
        pltpu.sync_copy(dst_hbm.at[pl.ds(base, _RM1)], dste)
        pltpu.async_copy(feats_hbm.at[srce], rows0.at[pl.ds(0, _RM1)],
                         semg0).wait()
        pltpu.sync_copy(rows0.at[pl.ds(0, _RM1)], acc_sp.at[dste], add=True)

    @pl.when(cid == 1)
    def _deg():
        pltpu.sync_copy(ones_hbm, rows0)
        pltpu.sync_copy(dst_hbm.at[pl.ds(e0, _CF)], dstv0)

        def body(jo, _):
            for p in range(2):
                c = 2 * jo + p

                @pl.when(c + 1 < _NF1)
                def _idx():
                    base2 = e0 + (c + 1) * _CF
                    pltpu.async_copy(dst_hbm.at[pl.ds(base2, _CF)],
                                     dring[(p + 1) % 2], semi0)

                pltpu.sync_copy(rows0, acc_sp.at[dring[p]], add=True)

                @pl.when(c + 1 < _NF1)
                def _w():
                    base2 = e0 + (c + 1) * _CF
                    pltpu.make_async_copy(dst_hbm.at[pl.ds(base2, _CF)],
                                          dring[(p + 1) % 2], semi0).wait()

            return 0

        lax.fori_loop(0, (_NF1 - 1) // 2, body, 0)

        pltpu.sync_copy(rows0, acc_sp.at[dring[0]], add=True)

        base = e0 + _NF1 * _CF
        pltpu.sync_copy(dst_hbm.at[pl.ds(base, _RM1)], dste)
        pltpu.sync_copy(rows0.at[pl.ds(0, _RM1)], acc_sp.at[dste], add=True)

    plsc.subcore_barrier()
    pltpu.sync_copy(acc_sp.at[pl.ds(n0, nn)], out_hbm.at[cid, pl.ds(n0, nn)])


def _segsum_deg(feats, src, dst):
    zrows = jnp.zeros((NP // NS, D), jnp.float32)
    ones = jnp.ones((_CF, D), jnp.float32)
    mesh = plsc.VectorSubcoreMesh(core_axis_name="c", subcore_axis_name="s")
    fn = functools.partial(
        pl.kernel,
        mesh=mesh,
        out_type=jax.ShapeDtypeStruct((NC, NP, D), jnp.float32),
        scratch_types=[
            pltpu.VMEM_SHARED((NP, D), jnp.float32),
            pltpu.VMEM((_CF,), jnp.int32),
            pltpu.VMEM((_CF, D), jnp.float32),
            pltpu.SemaphoreType.DMA,
            pltpu.SemaphoreType.DMA,
            pltpu.VMEM((_CF,), jnp.int32),
            pltpu.VMEM((_CF, D), jnp.float32),
            pltpu.SemaphoreType.DMA,
            pltpu.SemaphoreType.DMA,
            pltpu.VMEM((_CF,), jnp.int32),
            pltpu.VMEM((_CF,), jnp.int32),
            pltpu.VMEM((_CF,), jnp.int32),
            pltpu.VMEM((_CF,), jnp.int32),
            pltpu.VMEM((_RM1,), jnp.int32),
            pltpu.VMEM((_RM1,), jnp.int32),
        ],
    )(_segsum_deg_kernel)
    return fn(feats, src, dst, zrows, ones)


_CB = 176
_NFULL = 56
_CREM = 144


def _segsum_kernel(feats_hbm, src_hbm, dst_hbm, zrows_hbm, out_hbm, acc_sp,
                   srcv0, rows0, semg0, semi0, srcv1, rows1, semg1, semi1,
                   dstv0, dstv1, dstv2, dstv3, srce, dste):
    cid = lax.axis_index("c")
    sid = lax.axis_index("s")
    wid = cid * NS + sid
    n0 = sid * (NP // NS)
    nn = NP // NS
    e0 = wid * (E // NW)

    pltpu.sync_copy(zrows_hbm, acc_sp.at[pl.ds(n0, nn)])
    plsc.subcore_barrier()

    gbufs = ((srcv0, rows0, semg0, semi0), (srcv1, rows1, semg1, semi1))
    dring = (dstv0, dstv1, dstv2, dstv3)

    for c in range(2):
        srcv, rows, semg, _ = gbufs[c % 2]
        base = e0 + c * _CB
        pltpu.sync_copy(src_hbm.at[pl.ds(base, _CB)], srcv)
        pltpu.sync_copy(dst_hbm.at[pl.ds(base, _CB)], dring[c])
        pltpu.async_copy(feats_hbm.at[srcv], rows, semg)

    def body(jo, _):
        for q in range(4):
            c = 4 * jo + q
            b = q % 2
            srcv, rows, semg, semi = gbufs[b]
            pltpu.make_async_copy(feats_hbm.at[srcv], rows, semg).wait()

            @pl.when(c + 2 < _NFULL)
            def _idx():
                base2 = e0 + (c + 2) * _CB
                pltpu.async_copy(src_hbm.at[pl.ds(base2, _CB)], srcv, semi)
                pltpu.async_copy(dst_hbm.at[pl.ds(base2, _CB)],
                                 dring[(q + 2) % 4], semi)

            pltpu.sync_copy(rows, acc_sp.at[dring[q % 4]], add=True)

            @pl.when(c + 2 < _NFULL)
            def _fire():
                base2 = e0 + (c + 2) * _CB
                pltpu.make_async_copy(src_hbm.at[pl.ds(base2, _CB)], srcv,
                                      semi).wait()
                pltpu.make_async_copy(dst_hbm.at[pl.ds(base2, _CB)],
                                      dring[(q + 2) % 4], semi).wait()
                pltpu.async_copy(feats_hbm.at[srcv], rows, semg)

        return 0

    lax.fori_loop(0, _NFULL // 4, body, 0)

    base = e0 + _NFULL * _CB
    pltpu.sync_copy(src_hbm.at[pl.ds(base, _CREM)], srce)
    pltpu.sync_copy(dst_hbm.at[pl.ds(base, _CREM)], dste)
    pltpu.async_copy(feats_hbm.at[srce], rows0.at[pl.ds(0, _CREM)],
                     semg0).wait()
    pltpu.sync_copy(rows0.at[pl.ds(0, _CREM)], acc_sp.at[dste], add=True)

    plsc.subcore_barrier()
    pltpu.sync_copy(acc_sp.at[pl.ds(n0, nn)], out_hbm.at[cid, pl.ds(n0, nn)])


def _segsum(feats, src, dst):
    zrows = jnp.zeros((NP // NS, D), jnp.float32)
    mesh = plsc.VectorSubcoreMesh(core_axis_name="c", subcore_axis_name="s")
    fn = functools.partial(
        pl.kernel,
        mesh=mesh,
        out_type=jax.ShapeDtypeStruct((NC, NP, D), jnp.float32),
        scratch_types=[
            pltpu.VMEM_SHARED((NP, D), jnp.float32),
            pltpu.VMEM((_CB,), jnp.int32),
            pltpu.VMEM((_CB, D), jnp.float32),
            pltpu.SemaphoreType.DMA,
            pltpu.SemaphoreType.DMA,
            pltpu.VMEM((_CB,), jnp.int32),
            pltpu.VMEM((_CB, D), jnp.float32),
            pltpu.SemaphoreType.DMA,
            pltpu.SemaphoreType.DMA,
            pltpu.VMEM((_CB,), jnp.int32),
            pltpu.VMEM((_CB,), jnp.int32),
            pltpu.VMEM((_CB,), jnp.int32),
            pltpu.VMEM((_CB,), jnp.int32),
            pltpu.VMEM((_CREM,), jnp.int32),
            pltpu.VMEM((_CREM,), jnp.int32),
        ],
    )(_segsum_kernel)
    return fn(feats, src, dst, zrows)


_RB = 1280


def _layer1_body(pref, fref, wlref, wrref, bref, zref, invref):
    p = pref[...]
    inv = 1.0 / jnp.maximum(p[1, :, 0:1], 1.0)
    agg = p[0] * inv
    dn = (((1,), (1,)), ((), ()))
    h = (lax.dot_general(agg, wlref[...], dn,
                         preferred_element_type=jnp.float32)
         + lax.dot_general(fref[...], wrref[...], dn,
                           preferred_element_type=jnp.float32)
         + bref[...])
    zref[...] = jnp.maximum(h, 0.0)
    invref[...] = inv


def _tc_layer1(partials, feats, WlT, WrT, b2d):
    return pl.pallas_call(
        _layer1_body,
        grid=(NP // _RB,),
        in_specs=[
            pl.BlockSpec((NC, _RB, D), lambda i: (0, i, 0)),
            pl.BlockSpec((_RB, D), lambda i: (i, 0)),
            pl.BlockSpec((D, D), lambda i: (0, 0)),
            pl.BlockSpec((D, D), lambda i: (0, 0)),
            pl.BlockSpec((1, D), lambda i: (0, 0)),
        ],
        out_specs=[pl.BlockSpec((_RB, D), lambda i: (i, 0)),
                   pl.BlockSpec((_RB, 1), lambda i: (i, 0))],
        out_shape=[jax.ShapeDtypeStruct((NP, D), jnp.float32),
                   jax.ShapeDtypeStruct((NP, 1), jnp.float32)],
    )(partials, feats, WlT, WrT, b2d)


def _layer2_body(pref, invref, fref, wlref, wrref, bref, zref):
    p = pref[...]
    agg = (p[0] + p[1]) * invref[...]
    dn = (((1,), (1,)), ((), ()))
    h = (lax.dot_general(agg, wlref[...], dn,
                         preferred_element_type=jnp.float32)
         + lax.dot_general(fref[...], wrref[...], dn,
                           preferred_element_type=jnp.float32)
         + bref[...])
    zref[...] = h


def _tc_layer2(partials, inv_col, feats, WlT, WrT, b2d):
    return pl.pallas_call(
        _layer2_body,
        grid=(NP // _RB,),
        in_specs=[
            pl.BlockSpec((NC, _RB, D), lambda i: (0, i, 0)),
            pl.BlockSpec((_RB, 1), lambda i: (i, 0)),
            pl.BlockSpec((_RB, D), lambda i: (i, 0)),
            pl.BlockSpec((D, D), lambda i: (0, 0)),
            pl.BlockSpec((D, D), lambda i: (0, 0)),
            pl.BlockSpec((1, D), lambda i: (0, 0)),
        ],
        out_specs=pl.BlockSpec((_RB, D), lambda i: (i, 0)),
        out_shape=jax.ShapeDtypeStruct((NP, D), jnp.float32),
    )(partials, inv_col, feats, WlT, WrT, b2d)


_CE = 160
_NCH = L // _CE
LP = 200704


def _decode_kernel(z_hbm, ai_hbm, bi_hbm, out_hbm,
                   aidx0, bidx0, arows0, brows0, dots0, sema0, semb0, semd0,
                   aidx1, bidx1, arows1, brows1, dots1, sema1, semb1, semd1):
    cid = lax.axis_index("c")
    sid = lax.axis_index("s")
    wid = cid * NS + sid

    bufs = ((aidx0, bidx0, arows0, brows0, dots0, sema0, semb0, semd0),
            (aidx1, bidx1, arows1, brows1, dots1, sema1, semb1, semd1))

    def fire(j, b):
        ch = wid + j * NW

        @pl.when(ch < _NCH)
        def _f():
            aidx, bidx, arows, brows, dots, sema, semb, semd = bufs[b]
            base = ch * _CE
            pltpu.sync_copy(ai_hbm.at[pl.ds(base, _CE)], aidx)
            pltpu.sync_copy(bi_hbm.at[pl.ds(base, _CE)], bidx)
            pltpu.async_copy(z_hbm.at[aidx], arows, sema)
            pltpu.async_copy(z_hbm.at[bidx], brows, semb)

    fire(0, 0)
    fire(1, 1)

    def chunk_body(jo, _):
        for b in range(2):
            j = 2 * jo + b
            ch = wid + j * NW
            ch2 = wid + (j + 2) * NW

            @pl.when(ch < _NCH)
            def _do():
                aidx, bidx, arows, brows, dots, sema, semb, semd = bufs[b]
                base = ch * _CE
                pltpu.make_async_copy(z_hbm.at[aidx], arows, sema).wait()
                pltpu.make_async_copy(z_hbm.at[bidx], brows, semb).wait()

                @pl.when(j >= 2)
                def _wd():
                    pbase = (ch - 2 * NW) * _CE
                    pltpu.make_async_copy(
                        dots, out_hbm.at[pl.ds(pbase, _CE)], semd).wait()

                @pl.when(ch2 < _NCH)
                def _idx():
                    base2 = ch2 * _CE
                    pltpu.async_copy(ai_hbm.at[pl.ds(base2, _CE)], aidx, sema)
                    pltpu.async_copy(bi_hbm.at[pl.ds(base2, _CE)], bidx, semb)

                def pair_body(g, _):
                    i = g * 4
                    accs = [arows[i + u, pl.ds(0, 16)]
                            * brows[i + u, pl.ds(0, 16)] for u in range(4)]
                    for kk in range(1, D // 16):
                        for u in range(4):
                            accs[u] = accs[u] + (
                                arows[i + u, pl.ds(kk * 16, 16)]
                                * brows[i + u, pl.ds(kk * 16, 16)])
                    for u in range(4):
                        dots[i + u] = accs[u]
                    return 0

                lax.fori_loop(0, _CE // 4, pair_body, 0)
                pltpu.async_copy(dots, out_hbm.at[pl.ds(base, _CE)], semd)

                @pl.when(ch2 < _NCH)
                def _fire2():
                    base2 = ch2 * _CE
                    pltpu.make_async_copy(ai_hbm.at[pl.ds(base2, _CE)], aidx,
                                          sema).wait()
                    pltpu.make_async_copy(bi_hbm.at[pl.ds(base2, _CE)], bidx,
                                          semb).wait()
                    pltpu.async_copy(z_hbm.at[aidx], arows, sema)
                    pltpu.async_copy(z_hbm.at[bidx], brows, semb)

        return 0

    lax.fori_loop(0, (_NCH + NW - 1) // NW // 2, chunk_body, 0)

    nv = (_NCH - wid + NW - 1) // NW
    for b in range(2):
        jl = jnp.where((nv - 1) % 2 == b, nv - 1, nv - 2)
        lbase = (wid + jl * NW) * _CE
        dots = bufs[b][4]
        semd = bufs[b][7]
        pltpu.make_async_copy(dots, out_hbm.at[pl.ds(lbase, _CE)],
                              semd).wait()


def _decode_partial(z, ai, bi):
    mesh = plsc.VectorSubcoreMesh(core_axis_name="c", subcore_axis_name="s")
    buf_types = [
        pltpu.VMEM((_CE,), jnp.int32),
        pltpu.VMEM((_CE,), jnp.int32),
        pltpu.VMEM((_CE, D), jnp.float32),
        pltpu.VMEM((_CE, D), jnp.float32),
        pltpu.VMEM((_CE, 16), jnp.float32),
        pltpu.SemaphoreType.DMA,
        pltpu.SemaphoreType.DMA,
        pltpu.SemaphoreType.DMA,
    ]
    fn = functools.partial(
        pl.kernel,
        mesh=mesh,
        out_type=jax.ShapeDtypeStruct((LP, 16), jnp.float32),
        scratch_types=buf_types + buf_types,
    )(_decode_kernel)
    return fn(z, ai, bi)


_RF = 4096


def _fold_body(iref, oref):
    oref[...] = jnp.sum(iref[...], axis=1, keepdims=True)


def _fold16(dots16):
    return pl.pallas_call(
        _fold_body,
        grid=(LP // _RF,),
        in_specs=[pl.BlockSpec((_RF, 16), lambda i: (i, 0))],
        out_specs=pl.BlockSpec((_RF, 1), lambda i: (i, 0)),
        out_shape=jax.ShapeDtypeStruct((LP, 1), jnp.float32),
    )(dots16)


@jax.jit
def kernel(x, edge_index, edge_label_index, W1_l, b1, W1_r, W2_l, b2, W2_r):
    src = edge_index[0]
    dst = edge_index[1]
    xp = jnp.pad(x, ((0, NP - N), (0, 0)))

    p1 = _segsum_deg(xp, src, dst)
    z1, inv_col = _tc_layer1(p1, xp, W1_l, W1_r, b1.reshape(1, D))
    p2 = _segsum(z1, src, dst)
    z2 = _tc_layer2(p2, inv_col, z1, W2_l, W2_r, b2.reshape(1, D))

    dots16 = _decode_partial(z2, edge_label_index[0], edge_label_index[1])
    return _fold16(dots16).reshape(LP)[:L]
